# baseline (speedup 1.0000x reference)
import jax
import jax.numpy as jnp
import numpy as np
from jax import lax
from jax.experimental import pallas as pl
from jax.experimental.pallas import tpu as pltpu
from jax.experimental.pallas import tpu_sc as plsc

N = 10000
E = 320000
D = 128
DE = 16
H = 128
L = 3
C = 40

NC = 2
NS = 16
NW = NC * NS
CHUNK = 128
NCHUNKS = E // CHUNK
CPW = NCHUNKS // NW
NTAIL = NCHUNKS - CPW * NW
RPT = 624
RTAIL = N - RPT * NS
BE = 8000

SPW = 80
MPW = SPW // 4
NBLK = NW * MPW
NPAD = N + 16

_slots = []
for _w in range(NW):
    _sl = list(range(CPW * _w, CPW * _w + CPW))
    _sl.append(NW * CPW + _w if _w < NTAIL else -1)
    _sl.append(-1)
    _slots += _sl
_SLOT_IDS = np.asarray(_slots, np.int32)
_SLOT_MASK = _SLOT_IDS >= 0


def _tile_copy(s, src_ref, dst_ref):
    row0 = s * RPT
    pltpu.sync_copy(src_ref.at[pl.ds(row0, RPT), :],
                    dst_ref.at[pl.ds(row0, RPT), :])

    @pl.when(s == NS - 1)
    def _():
        pltpu.sync_copy(src_ref.at[pl.ds(RPT * NS, RTAIL), :],
                        dst_ref.at[pl.ds(RPT * NS, RTAIL), :])


def _spmv_body(table, sd, zinit, out,
               bidx0, bidx1, rows_v0, rows_v1, acc,
               rsem0, rsem1, bsem0, bsem1):
    c = lax.axis_index("c")
    s = lax.axis_index("s")
    wid = c * NS + s

    _tile_copy(s, zinit, acc)
    plsc.subcore_barrier()

    bx = (bidx0, bidx1)
    rv = (rows_v0, rows_v1)
    rsem = (rsem0, rsem1)
    bsem = (bsem0, bsem1)
    m0 = wid * MPW

    def bload(m, bb):
        pltpu.async_copy(sd.at[m], bx[bb], bsem[bb])

    def bwait(m, bb):
        pltpu.make_async_copy(sd.at[m], bx[bb], bsem[bb]).wait()

    def g_start(bb, t, b):
        pltpu.async_copy(table.at[bx[bb].at[2 * t]], rv[b], rsem[b])

    def g_wait(bb, t, b):
        pltpu.make_async_copy(table.at[bx[bb].at[2 * t]], rv[b], rsem[b]).wait()

    def scat(bb, t, b):
        pltpu.sync_copy(rv[b], acc.at[bx[bb].at[2 * t + 1]], add=True)

    pltpu.sync_copy(sd.at[m0], bidx0)
    bload(m0 + 1, 1)
    g_start(0, 0, 0)
    g_start(0, 1, 1)

    def body(i, _):
        m = m0 + 2 * i
        g_wait(0, 0, 0); scat(0, 0, 0); g_start(0, 2, 0)
        g_wait(0, 1, 1); scat(0, 1, 1); bwait(m + 1, 1); g_start(0, 3, 1)
        g_wait(0, 2, 0); scat(0, 2, 0); g_start(1, 0, 0)
        g_wait(0, 3, 1); scat(0, 3, 1); bload(m + 2, 0); g_start(1, 1, 1)
        g_wait(1, 0, 0); scat(1, 0, 0); g_start(1, 2, 0)
        g_wait(1, 1, 1); scat(1, 1, 1); bwait(m + 2, 0); g_start(1, 3, 1)
        g_wait(1, 2, 0); scat(1, 2, 0); g_start(0, 0, 0)
        g_wait(1, 3, 1); scat(1, 3, 1); bload(m + 3, 1); g_start(0, 1, 1)
        return 0

    lax.fori_loop(0, MPW // 2, body, 0)

    g_wait(0, 0, 0)
    g_wait(0, 1, 1)
    bwait(m0 + MPW + 1, 1)

    plsc.subcore_barrier()
    _tile_copy(s, acc, out.at[c])


_sc_spmv = pl.kernel(
    _spmv_body,
    out_type=jax.ShapeDtypeStruct((NC, N, D), jnp.float32),
    mesh=plsc.VectorSubcoreMesh(core_axis_name="c", subcore_axis_name="s"),
    scratch_types=[
        pltpu.VMEM((8, CHUNK), jnp.int32),
        pltpu.VMEM((8, CHUNK), jnp.int32),
        pltpu.VMEM((CHUNK, D), jnp.float32),
        pltpu.VMEM((CHUNK, D), jnp.float32),
        pltpu.VMEM_SHARED((NPAD, D), jnp.float32),
        pltpu.SemaphoreType.DMA,
        pltpu.SemaphoreType.DMA,
        pltpu.SemaphoreType.DMA,
        pltpu.SemaphoreType.DMA,
    ],
)


def _edge_scatter_body(eh_full, dst, zinit, out,
                       dst_v0, rows_v0, dst_v1, rows_v1, acc, sem0, sem1):
    c = lax.axis_index("c")
    s = lax.axis_index("s")
    wid = c * NS + s

    _tile_copy(s, zinit, acc)
    plsc.subcore_barrier()

    dv = (dst_v0, dst_v1)
    rv = (rows_v0, rows_v1)
    sems = (sem0, sem1)

    def start(j, b):
        base = j * CHUNK
        pltpu.sync_copy(dst.at[pl.ds(base, CHUNK)], dv[b])
        pltpu.async_copy(eh_full.at[pl.ds(base, CHUNK), :], rv[b], sems[b])

    def finish(j, b):
        base = j * CHUNK
        pltpu.make_async_copy(eh_full.at[pl.ds(base, CHUNK), :], rv[b],
                              sems[b]).wait()
        pltpu.sync_copy(rv[b], acc.at[dv[b]], add=True)

    j0 = wid * CPW
    start(j0, 0)
    start(j0 + 1, 1)

    def body(i, _):
        j = j0 + 2 * i
        finish(j, 0)
        start(j + 2, 0)
        finish(j + 1, 1)
        start(j + 3, 1)
        return 0

    lax.fori_loop(0, (CPW - 2) // 2, body, 0)
    finish(j0 + CPW - 2, 0)
    finish(j0 + CPW - 1, 1)

    @pl.when(wid < NTAIL)
    def _():
        start(NW * CPW + wid, 0)
        finish(NW * CPW + wid, 0)

    plsc.subcore_barrier()
    _tile_copy(s, acc, out.at[c])


_sc_edge_scatter = pl.kernel(
    _edge_scatter_body,
    out_type=jax.ShapeDtypeStruct((NC, N, D), jnp.float32),
    mesh=plsc.VectorSubcoreMesh(core_axis_name="c", subcore_axis_name="s"),
    scratch_types=[
        pltpu.VMEM((CHUNK,), jnp.int32),
        pltpu.VMEM((CHUNK, D), jnp.float32),
        pltpu.VMEM((CHUNK,), jnp.int32),
        pltpu.VMEM((CHUNK, D), jnp.float32),
        pltpu.VMEM_SHARED((N, D), jnp.float32),
        pltpu.SemaphoreType.DMA,
        pltpu.SemaphoreType.DMA,
    ],
)


_DN = (((1,), (1,)), ((), ()))


def _eh_body(e_ref, we_ref, be_ref, o_ref):
    o_ref[...] = lax.dot_general(e_ref[...], we_ref[...], _DN,
                                 preferred_element_type=jnp.float32) + be_ref[...]


def _tc_eh(e, W_edge, b_edge):
    return pl.pallas_call(
        _eh_body,
        grid=(E // BE,),
        in_specs=[pl.BlockSpec((BE, DE), lambda i: (i, 0)),
                  pl.BlockSpec((H, DE), lambda i: (0, 0)),
                  pl.BlockSpec((1, H), lambda i: (0, 0))],
        out_specs=pl.BlockSpec((BE, H), lambda i: (i, 0)),
        out_shape=jax.ShapeDtypeStruct((E, H), jnp.float32),
    )(e, W_edge, b_edge.reshape(1, H))


def _h_body(x_ref, wn_ref, bn_ref, h_ref):
    h_ref[...] = lax.dot_general(x_ref[...], wn_ref[...], _DN,
                                 preferred_element_type=jnp.float32) + bn_ref[...]


def _tc_h(x, W_node, b_node):
    return pl.pallas_call(
        _h_body,
        out_shape=jax.ShapeDtypeStruct((N, H), jnp.float32),
    )(x, W_node, b_node.reshape(1, H))


def _layer0_body(p_ref, pe_ref, tin_ref, w1_ref, b1_ref, g1_ref, be1_ref,
                 w2_ref, b2_ref, hs_ref, tout_ref, eh_ref):
    eh = pe_ref[0] + pe_ref[1]
    eh_ref[...] = eh
    aggr = p_ref[0] + p_ref[1] + eh
    z = lax.dot_general(aggr, w1_ref[...], _DN,
                        preferred_element_type=jnp.float32) + b1_ref[...]
    m = jnp.mean(z, axis=0, keepdims=True)
    v = jnp.mean((z - m) ** 2, axis=0, keepdims=True)
    z = g1_ref[...] * (z - m) / jnp.sqrt(v + 1e-5) + be1_ref[...]
    z = jnp.maximum(z, 0.0)
    z = lax.dot_general(z, w2_ref[...], _DN,
                        preferred_element_type=jnp.float32) + b2_ref[...]
    hs = jnp.where(z > 0, z, 0.2 * z)
    hs_ref[...] = hs
    tout_ref[...] = tin_ref[...] + hs


def _tc_layer0(p, pe, total, W1l, b1l, g1l, be1l, W2l, b2l):
    return pl.pallas_call(
        _layer0_body,
        out_shape=[jax.ShapeDtypeStruct((N, H), jnp.float32),
                   jax.ShapeDtypeStruct((N, H), jnp.float32),
                   jax.ShapeDtypeStruct((N, H), jnp.float32)],
    )(p, pe, total, W1l, b1l.reshape(1, 2 * H), g1l.reshape(1, 2 * H),
      be1l.reshape(1, 2 * H), W2l, b2l.reshape(1, H))


def _layer_body(p_ref, eh_ref, tin_ref, w1_ref, b1_ref, g1_ref, be1_ref,
                w2_ref, b2_ref, hs_ref, tout_ref):
    aggr = p_ref[0] + p_ref[1] + eh_ref[...]
    z = lax.dot_general(aggr, w1_ref[...], _DN,
                        preferred_element_type=jnp.float32) + b1_ref[...]
    m = jnp.mean(z, axis=0, keepdims=True)
    v = jnp.mean((z - m) ** 2, axis=0, keepdims=True)
    z = g1_ref[...] * (z - m) / jnp.sqrt(v + 1e-5) + be1_ref[...]
    z = jnp.maximum(z, 0.0)
    z = lax.dot_general(z, w2_ref[...], _DN,
                        preferred_element_type=jnp.float32) + b2_ref[...]
    hs = jnp.where(z > 0, z, 0.2 * z)
    hs_ref[...] = hs
    tout_ref[...] = tin_ref[...] + hs


def _tc_layer(p, eh, total, W1l, b1l, g1l, be1l, W2l, b2l):
    return pl.pallas_call(
        _layer_body,
        out_shape=[jax.ShapeDtypeStruct((N, H), jnp.float32),
                   jax.ShapeDtypeStruct((N, H), jnp.float32)],
    )(p, eh, total, W1l, b1l.reshape(1, 2 * H), g1l.reshape(1, 2 * H),
      be1l.reshape(1, 2 * H), W2l, b2l.reshape(1, H))


def _head_body(t_ref, wo1_ref, bo1_ref, go_ref, beo_ref, ap_ref, wo2_ref,
               bo2_ref, o_ref):
    z = lax.dot_general(t_ref[...], wo1_ref[...], _DN,
                        preferred_element_type=jnp.float32) + bo1_ref[...]
    m = jnp.mean(z, axis=0, keepdims=True)
    v = jnp.mean((z - m) ** 2, axis=0, keepdims=True)
    z = go_ref[...] * (z - m) / jnp.sqrt(v + 1e-5) + beo_ref[...]
    z = jnp.where(z > 0, z, ap_ref[...] * z)
    o_ref[...] = lax.dot_general(z, wo2_ref[...], _DN,
                                 preferred_element_type=jnp.float32) + bo2_ref[...]


def _tc_head(total, Wo1, bo1, go, beo, a_prelu, Wo2, bo2):
    return pl.pallas_call(
        _head_body,
        out_shape=jax.ShapeDtypeStruct((N, C), jnp.float32),
    )(total, Wo1, bo1.reshape(1, 2 * H), go.reshape(1, 2 * H),
      beo.reshape(1, 2 * H), a_prelu.reshape(1, 1), Wo2, bo2.reshape(1, C))


def kernel(x, edge_index, e, W_node, b_node, W_edge, b_edge, W1, b1, g1, be1,
           W2, b2, Wo1, bo1, go, beo, a_prelu, Wo2, bo2):
    src = edge_index[0]
    dst = edge_index[1]

    ids = jnp.asarray(np.maximum(_SLOT_IDS, 0))
    mask = jnp.asarray(_SLOT_MASK)[:, None]
    scrap = N + (jnp.arange(CHUNK, dtype=jnp.int32) % (NPAD - N))
    src_s = jnp.where(mask, src.reshape(NCHUNKS, CHUNK)[ids], 0)
    dst_s = jnp.where(mask, dst.reshape(NCHUNKS, CHUNK)[ids], scrap[None, :])
    sd = jnp.stack([src_s, dst_s], axis=1).reshape(NBLK, 8, CHUNK)
    sd = jnp.concatenate([sd, jnp.zeros((4, 8, CHUNK), jnp.int32)])

    zin_d = jnp.zeros((N, D), jnp.float32)

    h = _tc_h(x, W_node, b_node)
    p = _sc_spmv(h, sd, zin_d)
    eh_full = _tc_eh(e, W_edge, b_edge)
    pe = _sc_edge_scatter(eh_full, dst, zin_d)

    hs, total, eh = _tc_layer0(p, pe, h, W1[0], b1[0], g1[0], be1[0],
                               W2[0], b2[0])
    for l in range(1, L):
        p = _sc_spmv(hs, sd, zin_d)
        hs, total = _tc_layer(p, eh, total, W1[l], b1[l], g1[l], be1[l],
                              W2[l], b2[l])

    return _tc_head(total, Wo1, bo1, go, beo, a_prelu, Wo2, bo2)


# speedup vs baseline: 2.1024x; 2.1024x over previous
import jax
import jax.numpy as jnp
from jax import lax
from jax.experimental import pallas as pl
from jax.experimental.pallas import tpu as pltpu
from jax.experimental.pallas import tpu_sc as plsc

N = 10000
E = 320000
D = 128
DE = 16
H = 128
L = 3
C = 40

NC = 2
NS = 16
NW = NC * NS
CHUNK = 128
NCHUNKS = E // CHUNK
CPW = NCHUNKS // NW
NTAIL = NCHUNKS - CPW * NW
RPT = 624
RTAIL = N - RPT * NS
BE = 8000


def _tile_copy(s, src_ref, dst_ref):
    row0 = s * RPT
    pltpu.sync_copy(src_ref.at[pl.ds(row0, RPT), :],
                    dst_ref.at[pl.ds(row0, RPT), :])

    @pl.when(s == NS - 1)
    def _():
        pltpu.sync_copy(src_ref.at[pl.ds(RPT * NS, RTAIL), :],
                        dst_ref.at[pl.ds(RPT * NS, RTAIL), :])


def _spmv_body(table, src, dst, zinit, out,
               src_v0, dst_v0, rows_v0, src_v1, dst_v1, rows_v1,
               acc, sem0, sem1, ssem0, ssem1):
    c = lax.axis_index("c")
    s = lax.axis_index("s")
    wid = c * NS + s

    _tile_copy(s, zinit, acc)
    plsc.subcore_barrier()

    sv = (src_v0, src_v1)
    dv = (dst_v0, dst_v1)
    rv = (rows_v0, rows_v1)
    sems = (sem0, sem1)
    ssems = (ssem0, ssem1)

    def start(j, b):
        base = j * CHUNK
        pltpu.sync_copy(src.at[pl.ds(base, CHUNK)], sv[b])
        pltpu.sync_copy(dst.at[pl.ds(base, CHUNK)], dv[b])
        pltpu.async_copy(table.at[sv[b]], rv[b], sems[b])

    def mid(b):
        pltpu.make_async_copy(table.at[sv[b]], rv[b], sems[b]).wait()
        pltpu.async_copy(rv[b], acc.at[dv[b]], ssems[b], add=True)

    def sfin(b):
        pltpu.make_async_copy(rv[b], acc.at[dv[b]], ssems[b]).wait()

    j0 = wid * CPW
    start(j0, 0)
    start(j0 + 1, 1)

    def body(i, _):
        j = j0 + 2 * i
        mid(0)
        mid(1)
        sfin(0)
        start(j + 2, 0)
        sfin(1)
        start(j + 3, 1)
        return 0

    lax.fori_loop(0, (CPW - 2) // 2, body, 0)
    mid(0)
    mid(1)
    sfin(0)
    sfin(1)

    @pl.when(wid < NTAIL)
    def _():
        start(NW * CPW + wid, 0)
        mid(0)
        sfin(0)

    plsc.subcore_barrier()
    _tile_copy(s, acc, out.at[c])


_sc_spmv = pl.kernel(
    _spmv_body,
    out_type=jax.ShapeDtypeStruct((NC, N, D), jnp.float32),
    mesh=plsc.VectorSubcoreMesh(core_axis_name="c", subcore_axis_name="s"),
    scratch_types=[
        pltpu.VMEM((CHUNK,), jnp.int32),
        pltpu.VMEM((CHUNK,), jnp.int32),
        pltpu.VMEM((CHUNK, D), jnp.float32),
        pltpu.VMEM((CHUNK,), jnp.int32),
        pltpu.VMEM((CHUNK,), jnp.int32),
        pltpu.VMEM((CHUNK, D), jnp.float32),
        pltpu.VMEM_SHARED((N, D), jnp.float32),
        pltpu.SemaphoreType.DMA,
        pltpu.SemaphoreType.DMA,
        pltpu.SemaphoreType.DMA,
        pltpu.SemaphoreType.DMA,
    ],
)


def _edge_scatter_body(eh_full, dst, zinit, out,
                       dst_v0, rows_v0, dst_v1, rows_v1, acc, sem0, sem1):
    c = lax.axis_index("c")
    s = lax.axis_index("s")
    wid = c * NS + s

    _tile_copy(s, zinit, acc)
    plsc.subcore_barrier()

    dv = (dst_v0, dst_v1)
    rv = (rows_v0, rows_v1)
    sems = (sem0, sem1)

    def start(j, b):
        base = j * CHUNK
        pltpu.sync_copy(dst.at[pl.ds(base, CHUNK)], dv[b])
        pltpu.async_copy(eh_full.at[pl.ds(base, CHUNK), :], rv[b], sems[b])

    def finish(j, b):
        base = j * CHUNK
        pltpu.make_async_copy(eh_full.at[pl.ds(base, CHUNK), :], rv[b],
                              sems[b]).wait()
        pltpu.sync_copy(rv[b], acc.at[dv[b]], add=True)

    j0 = wid * CPW
    start(j0, 0)
    start(j0 + 1, 1)

    def body(i, _):
        j = j0 + 2 * i
        finish(j, 0)
        start(j + 2, 0)
        finish(j + 1, 1)
        start(j + 3, 1)
        return 0

    lax.fori_loop(0, (CPW - 2) // 2, body, 0)
    finish(j0 + CPW - 2, 0)
    finish(j0 + CPW - 1, 1)

    @pl.when(wid < NTAIL)
    def _():
        start(NW * CPW + wid, 0)
        finish(NW * CPW + wid, 0)

    plsc.subcore_barrier()
    _tile_copy(s, acc, out.at[c])


_sc_edge_scatter = pl.kernel(
    _edge_scatter_body,
    out_type=jax.ShapeDtypeStruct((NC, N, D), jnp.float32),
    mesh=plsc.VectorSubcoreMesh(core_axis_name="c", subcore_axis_name="s"),
    scratch_types=[
        pltpu.VMEM((CHUNK,), jnp.int32),
        pltpu.VMEM((CHUNK, D), jnp.float32),
        pltpu.VMEM((CHUNK,), jnp.int32),
        pltpu.VMEM((CHUNK, D), jnp.float32),
        pltpu.VMEM_SHARED((N, D), jnp.float32),
        pltpu.SemaphoreType.DMA,
        pltpu.SemaphoreType.DMA,
    ],
)


_DN = (((1,), (1,)), ((), ()))


def _eh_body(e_ref, we_ref, be_ref, o_ref):
    o_ref[...] = lax.dot_general(e_ref[...], we_ref[...], _DN,
                                 preferred_element_type=jnp.float32) + be_ref[...]


def _tc_eh(e, W_edge, b_edge):
    return pl.pallas_call(
        _eh_body,
        grid=(E // BE,),
        in_specs=[pl.BlockSpec((BE, DE), lambda i: (i, 0)),
                  pl.BlockSpec((H, DE), lambda i: (0, 0)),
                  pl.BlockSpec((1, H), lambda i: (0, 0))],
        out_specs=pl.BlockSpec((BE, H), lambda i: (i, 0)),
        out_shape=jax.ShapeDtypeStruct((E, H), jnp.float32),
    )(e, W_edge, b_edge.reshape(1, H))


def _h_body(x_ref, wn_ref, bn_ref, h_ref):
    h_ref[...] = lax.dot_general(x_ref[...], wn_ref[...], _DN,
                                 preferred_element_type=jnp.float32) + bn_ref[...]


def _tc_h(x, W_node, b_node):
    return pl.pallas_call(
        _h_body,
        out_shape=jax.ShapeDtypeStruct((N, H), jnp.float32),
    )(x, W_node, b_node.reshape(1, H))


def _layer0_body(p_ref, pe_ref, tin_ref, w1_ref, b1_ref, g1_ref, be1_ref,
                 w2_ref, b2_ref, hs_ref, tout_ref, eh_ref):
    eh = pe_ref[0] + pe_ref[1]
    eh_ref[...] = eh
    aggr = p_ref[0] + p_ref[1] + eh
    z = lax.dot_general(aggr, w1_ref[...], _DN,
                        preferred_element_type=jnp.float32) + b1_ref[...]
    m = jnp.mean(z, axis=0, keepdims=True)
    v = jnp.mean((z - m) ** 2, axis=0, keepdims=True)
    z = g1_ref[...] * (z - m) / jnp.sqrt(v + 1e-5) + be1_ref[...]
    z = jnp.maximum(z, 0.0)
    z = lax.dot_general(z, w2_ref[...], _DN,
                        preferred_element_type=jnp.float32) + b2_ref[...]
    hs = jnp.where(z > 0, z, 0.2 * z)
    hs_ref[...] = hs
    tout_ref[...] = tin_ref[...] + hs


def _tc_layer0(p, pe, total, W1l, b1l, g1l, be1l, W2l, b2l):
    return pl.pallas_call(
        _layer0_body,
        out_shape=[jax.ShapeDtypeStruct((N, H), jnp.float32),
                   jax.ShapeDtypeStruct((N, H), jnp.float32),
                   jax.ShapeDtypeStruct((N, H), jnp.float32)],
    )(p, pe, total, W1l, b1l.reshape(1, 2 * H), g1l.reshape(1, 2 * H),
      be1l.reshape(1, 2 * H), W2l, b2l.reshape(1, H))


def _layer_body(p_ref, eh_ref, tin_ref, w1_ref, b1_ref, g1_ref, be1_ref,
                w2_ref, b2_ref, hs_ref, tout_ref):
    aggr = p_ref[0] + p_ref[1] + eh_ref[...]
    z = lax.dot_general(aggr, w1_ref[...], _DN,
                        preferred_element_type=jnp.float32) + b1_ref[...]
    m = jnp.mean(z, axis=0, keepdims=True)
    v = jnp.mean((z - m) ** 2, axis=0, keepdims=True)
    z = g1_ref[...] * (z - m) / jnp.sqrt(v + 1e-5) + be1_ref[...]
    z = jnp.maximum(z, 0.0)
    z = lax.dot_general(z, w2_ref[...], _DN,
                        preferred_element_type=jnp.float32) + b2_ref[...]
    hs = jnp.where(z > 0, z, 0.2 * z)
    hs_ref[...] = hs
    tout_ref[...] = tin_ref[...] + hs


def _tc_layer(p, eh, total, W1l, b1l, g1l, be1l, W2l, b2l):
    return pl.pallas_call(
        _layer_body,
        out_shape=[jax.ShapeDtypeStruct((N, H), jnp.float32),
                   jax.ShapeDtypeStruct((N, H), jnp.float32)],
    )(p, eh, total, W1l, b1l.reshape(1, 2 * H), g1l.reshape(1, 2 * H),
      be1l.reshape(1, 2 * H), W2l, b2l.reshape(1, H))


def _head_body(t_ref, wo1_ref, bo1_ref, go_ref, beo_ref, ap_ref, wo2_ref,
               bo2_ref, o_ref):
    z = lax.dot_general(t_ref[...], wo1_ref[...], _DN,
                        preferred_element_type=jnp.float32) + bo1_ref[...]
    m = jnp.mean(z, axis=0, keepdims=True)
    v = jnp.mean((z - m) ** 2, axis=0, keepdims=True)
    z = go_ref[...] * (z - m) / jnp.sqrt(v + 1e-5) + beo_ref[...]
    z = jnp.where(z > 0, z, ap_ref[...] * z)
    o_ref[...] = lax.dot_general(z, wo2_ref[...], _DN,
                                 preferred_element_type=jnp.float32) + bo2_ref[...]


def _tc_head(total, Wo1, bo1, go, beo, a_prelu, Wo2, bo2):
    return pl.pallas_call(
        _head_body,
        out_shape=jax.ShapeDtypeStruct((N, C), jnp.float32),
    )(total, Wo1, bo1.reshape(1, 2 * H), go.reshape(1, 2 * H),
      beo.reshape(1, 2 * H), a_prelu.reshape(1, 1), Wo2, bo2.reshape(1, C))


def kernel(x, edge_index, e, W_node, b_node, W_edge, b_edge, W1, b1, g1, be1,
           W2, b2, Wo1, bo1, go, beo, a_prelu, Wo2, bo2):
    src = edge_index[0]
    dst = edge_index[1]

    zin_d = jnp.zeros((N, D), jnp.float32)

    h = _tc_h(x, W_node, b_node)
    p = _sc_spmv(h, src, dst, zin_d)
    eh_full = _tc_eh(e, W_edge, b_edge)
    pe = _sc_edge_scatter(eh_full, dst, zin_d)

    hs, total, eh = _tc_layer0(p, pe, h, W1[0], b1[0], g1[0], be1[0],
                               W2[0], b2[0])
    for l in range(1, L):
        p = _sc_spmv(hs, src, dst, zin_d)
        hs, total = _tc_layer(p, eh, total, W1[l], b1[l], g1[l], be1[l],
                              W2[l], b2[l])

    return _tc_head(total, Wo1, bo1, go, beo, a_prelu, Wo2, bo2)


# speedup vs baseline: 2.1536x; 1.0243x over previous
import jax
import jax.numpy as jnp
from jax import lax
from jax.experimental import pallas as pl
from jax.experimental.pallas import tpu as pltpu
from jax.experimental.pallas import tpu_sc as plsc

N = 10000
E = 320000
D = 128
DE = 16
H = 128
L = 3
C = 40

NC = 2
NS = 16
NW = NC * NS
CHUNK = 128
NCHUNKS = E // CHUNK
CPW = NCHUNKS // NW
NTAIL = NCHUNKS - CPW * NW
RPT = 624
RTAIL = N - RPT * NS
BE = 8000


def _tile_copy(s, src_ref, dst_ref):
    row0 = s * RPT
    pltpu.sync_copy(src_ref.at[pl.ds(row0, RPT), :],
                    dst_ref.at[pl.ds(row0, RPT), :])

    @pl.when(s == NS - 1)
    def _():
        pltpu.sync_copy(src_ref.at[pl.ds(RPT * NS, RTAIL), :],
                        dst_ref.at[pl.ds(RPT * NS, RTAIL), :])


def _spmv_body(table, src, dst, zinit, out,
               src_v0, dst_v0, rows_v0, src_v1, dst_v1, rows_v1,
               src_v2, dst_v2, rows_v2, acc, sem0, sem1, sem2):
    c = lax.axis_index("c")
    s = lax.axis_index("s")
    wid = c * NS + s

    _tile_copy(s, zinit, acc)
    plsc.subcore_barrier()

    sv = (src_v0, src_v1, src_v2)
    dv = (dst_v0, dst_v1, dst_v2)
    rv = (rows_v0, rows_v1, rows_v2)
    sems = (sem0, sem1, sem2)

    def start(j, b):
        base = j * CHUNK
        pltpu.sync_copy(src.at[pl.ds(base, CHUNK)], sv[b])
        pltpu.sync_copy(dst.at[pl.ds(base, CHUNK)], dv[b])
        pltpu.async_copy(table.at[sv[b]], rv[b], sems[b])

    def finish(b):
        pltpu.make_async_copy(table.at[sv[b]], rv[b], sems[b]).wait()
        pltpu.sync_copy(rv[b], acc.at[dv[b]], add=True)

    j0 = wid * CPW
    start(j0, 0)
    start(j0 + 1, 1)
    start(j0 + 2, 2)

    def body(i, _):
        j = j0 + 3 * i
        finish(0)
        start(j + 3, 0)
        finish(1)
        start(j + 4, 1)
        finish(2)
        start(j + 5, 2)
        return 0

    lax.fori_loop(0, (CPW - 3) // 3, body, 0)
    finish(0)
    finish(1)
    finish(2)

    @pl.when(wid < NTAIL)
    def _():
        start(NW * CPW + wid, 0)
        finish(0)

    plsc.subcore_barrier()
    _tile_copy(s, acc, out.at[c])


_sc_spmv = pl.kernel(
    _spmv_body,
    out_type=jax.ShapeDtypeStruct((NC, N, D), jnp.float32),
    mesh=plsc.VectorSubcoreMesh(core_axis_name="c", subcore_axis_name="s"),
    scratch_types=[
        pltpu.VMEM((CHUNK,), jnp.int32),
        pltpu.VMEM((CHUNK,), jnp.int32),
        pltpu.VMEM((CHUNK, D), jnp.float32),
        pltpu.VMEM((CHUNK,), jnp.int32),
        pltpu.VMEM((CHUNK,), jnp.int32),
        pltpu.VMEM((CHUNK, D), jnp.float32),
        pltpu.VMEM((CHUNK,), jnp.int32),
        pltpu.VMEM((CHUNK,), jnp.int32),
        pltpu.VMEM((CHUNK, D), jnp.float32),
        pltpu.VMEM_SHARED((N, D), jnp.float32),
        pltpu.SemaphoreType.DMA,
        pltpu.SemaphoreType.DMA,
        pltpu.SemaphoreType.DMA,
    ],
)


def _edge_scatter_body(eh_full, dst, zinit, out,
                       dst_v0, rows_v0, dst_v1, rows_v1, acc, sem0, sem1):
    c = lax.axis_index("c")
    s = lax.axis_index("s")
    wid = c * NS + s

    _tile_copy(s, zinit, acc)
    plsc.subcore_barrier()

    dv = (dst_v0, dst_v1)
    rv = (rows_v0, rows_v1)
    sems = (sem0, sem1)

    def start(j, b):
        base = j * CHUNK
        pltpu.sync_copy(dst.at[pl.ds(base, CHUNK)], dv[b])
        pltpu.async_copy(eh_full.at[pl.ds(base, CHUNK), :], rv[b], sems[b])

    def finish(j, b):
        base = j * CHUNK
        pltpu.make_async_copy(eh_full.at[pl.ds(base, CHUNK), :], rv[b],
                              sems[b]).wait()
        pltpu.sync_copy(rv[b], acc.at[dv[b]], add=True)

    j0 = wid * CPW
    start(j0, 0)
    start(j0 + 1, 1)

    def body(i, _):
        j = j0 + 2 * i
        finish(j, 0)
        start(j + 2, 0)
        finish(j + 1, 1)
        start(j + 3, 1)
        return 0

    lax.fori_loop(0, (CPW - 2) // 2, body, 0)
    finish(j0 + CPW - 2, 0)
    finish(j0 + CPW - 1, 1)

    @pl.when(wid < NTAIL)
    def _():
        start(NW * CPW + wid, 0)
        finish(NW * CPW + wid, 0)

    plsc.subcore_barrier()
    _tile_copy(s, acc, out.at[c])


_sc_edge_scatter = pl.kernel(
    _edge_scatter_body,
    out_type=jax.ShapeDtypeStruct((NC, N, D), jnp.float32),
    mesh=plsc.VectorSubcoreMesh(core_axis_name="c", subcore_axis_name="s"),
    scratch_types=[
        pltpu.VMEM((CHUNK,), jnp.int32),
        pltpu.VMEM((CHUNK, D), jnp.float32),
        pltpu.VMEM((CHUNK,), jnp.int32),
        pltpu.VMEM((CHUNK, D), jnp.float32),
        pltpu.VMEM_SHARED((N, D), jnp.float32),
        pltpu.SemaphoreType.DMA,
        pltpu.SemaphoreType.DMA,
    ],
)


_DN = (((1,), (1,)), ((), ()))


def _eh_body(e_ref, we_ref, be_ref, o_ref):
    o_ref[...] = lax.dot_general(e_ref[...], we_ref[...], _DN,
                                 preferred_element_type=jnp.float32) + be_ref[...]


def _tc_eh(e, W_edge, b_edge):
    return pl.pallas_call(
        _eh_body,
        grid=(E // BE,),
        in_specs=[pl.BlockSpec((BE, DE), lambda i: (i, 0)),
                  pl.BlockSpec((H, DE), lambda i: (0, 0)),
                  pl.BlockSpec((1, H), lambda i: (0, 0))],
        out_specs=pl.BlockSpec((BE, H), lambda i: (i, 0)),
        out_shape=jax.ShapeDtypeStruct((E, H), jnp.float32),
    )(e, W_edge, b_edge.reshape(1, H))


def _h_body(x_ref, wn_ref, bn_ref, h_ref):
    h_ref[...] = lax.dot_general(x_ref[...], wn_ref[...], _DN,
                                 preferred_element_type=jnp.float32) + bn_ref[...]


def _tc_h(x, W_node, b_node):
    return pl.pallas_call(
        _h_body,
        out_shape=jax.ShapeDtypeStruct((N, H), jnp.float32),
    )(x, W_node, b_node.reshape(1, H))


def _layer0_body(p_ref, pe_ref, tin_ref, w1_ref, b1_ref, g1_ref, be1_ref,
                 w2_ref, b2_ref, hs_ref, tout_ref, eh_ref):
    eh = pe_ref[0] + pe_ref[1]
    eh_ref[...] = eh
    aggr = p_ref[0] + p_ref[1] + eh
    z = lax.dot_general(aggr, w1_ref[...], _DN,
                        preferred_element_type=jnp.float32) + b1_ref[...]
    m = jnp.mean(z, axis=0, keepdims=True)
    v = jnp.mean((z - m) ** 2, axis=0, keepdims=True)
    z = g1_ref[...] * (z - m) / jnp.sqrt(v + 1e-5) + be1_ref[...]
    z = jnp.maximum(z, 0.0)
    z = lax.dot_general(z, w2_ref[...], _DN,
                        preferred_element_type=jnp.float32) + b2_ref[...]
    hs = jnp.where(z > 0, z, 0.2 * z)
    hs_ref[...] = hs
    tout_ref[...] = tin_ref[...] + hs


def _tc_layer0(p, pe, total, W1l, b1l, g1l, be1l, W2l, b2l):
    return pl.pallas_call(
        _layer0_body,
        out_shape=[jax.ShapeDtypeStruct((N, H), jnp.float32),
                   jax.ShapeDtypeStruct((N, H), jnp.float32),
                   jax.ShapeDtypeStruct((N, H), jnp.float32)],
    )(p, pe, total, W1l, b1l.reshape(1, 2 * H), g1l.reshape(1, 2 * H),
      be1l.reshape(1, 2 * H), W2l, b2l.reshape(1, H))


def _layer_body(p_ref, eh_ref, tin_ref, w1_ref, b1_ref, g1_ref, be1_ref,
                w2_ref, b2_ref, hs_ref, tout_ref):
    aggr = p_ref[0] + p_ref[1] + eh_ref[...]
    z = lax.dot_general(aggr, w1_ref[...], _DN,
                        preferred_element_type=jnp.float32) + b1_ref[...]
    m = jnp.mean(z, axis=0, keepdims=True)
    v = jnp.mean((z - m) ** 2, axis=0, keepdims=True)
    z = g1_ref[...] * (z - m) / jnp.sqrt(v + 1e-5) + be1_ref[...]
    z = jnp.maximum(z, 0.0)
    z = lax.dot_general(z, w2_ref[...], _DN,
                        preferred_element_type=jnp.float32) + b2_ref[...]
    hs = jnp.where(z > 0, z, 0.2 * z)
    hs_ref[...] = hs
    tout_ref[...] = tin_ref[...] + hs


def _tc_layer(p, eh, total, W1l, b1l, g1l, be1l, W2l, b2l):
    return pl.pallas_call(
        _layer_body,
        out_shape=[jax.ShapeDtypeStruct((N, H), jnp.float32),
                   jax.ShapeDtypeStruct((N, H), jnp.float32)],
    )(p, eh, total, W1l, b1l.reshape(1, 2 * H), g1l.reshape(1, 2 * H),
      be1l.reshape(1, 2 * H), W2l, b2l.reshape(1, H))


def _head_body(t_ref, wo1_ref, bo1_ref, go_ref, beo_ref, ap_ref, wo2_ref,
               bo2_ref, o_ref):
    z = lax.dot_general(t_ref[...], wo1_ref[...], _DN,
                        preferred_element_type=jnp.float32) + bo1_ref[...]
    m = jnp.mean(z, axis=0, keepdims=True)
    v = jnp.mean((z - m) ** 2, axis=0, keepdims=True)
    z = go_ref[...] * (z - m) / jnp.sqrt(v + 1e-5) + beo_ref[...]
    z = jnp.where(z > 0, z, ap_ref[...] * z)
    o_ref[...] = lax.dot_general(z, wo2_ref[...], _DN,
                                 preferred_element_type=jnp.float32) + bo2_ref[...]


def _tc_head(total, Wo1, bo1, go, beo, a_prelu, Wo2, bo2):
    return pl.pallas_call(
        _head_body,
        out_shape=jax.ShapeDtypeStruct((N, C), jnp.float32),
    )(total, Wo1, bo1.reshape(1, 2 * H), go.reshape(1, 2 * H),
      beo.reshape(1, 2 * H), a_prelu.reshape(1, 1), Wo2, bo2.reshape(1, C))


def kernel(x, edge_index, e, W_node, b_node, W_edge, b_edge, W1, b1, g1, be1,
           W2, b2, Wo1, bo1, go, beo, a_prelu, Wo2, bo2):
    src = edge_index[0]
    dst = edge_index[1]

    zin_d = jnp.zeros((N, D), jnp.float32)

    h = _tc_h(x, W_node, b_node)
    p = _sc_spmv(h, src, dst, zin_d)
    eh_full = _tc_eh(e, W_edge, b_edge)
    pe = _sc_edge_scatter(eh_full, dst, zin_d)

    hs, total, eh = _tc_layer0(p, pe, h, W1[0], b1[0], g1[0], be1[0],
                               W2[0], b2[0])
    for l in range(1, L):
        p = _sc_spmv(hs, src, dst, zin_d)
        hs, total = _tc_layer(p, eh, total, W1[l], b1[l], g1[l], be1[l],
                              W2[l], b2[l])

    return _tc_head(total, Wo1, bo1, go, beo, a_prelu, Wo2, bo2)


# speedup vs baseline: 2.1637x; 1.0047x over previous
import jax
import jax.numpy as jnp
from jax import lax
from jax.experimental import pallas as pl
from jax.experimental.pallas import tpu as pltpu
from jax.experimental.pallas import tpu_sc as plsc

N = 10000
E = 320000
D = 128
DE = 16
H = 128
L = 3
C = 40

NC = 2
NS = 16
NW = NC * NS
CHUNK = 128
NCHUNKS = E // CHUNK
CPW = NCHUNKS // NW
NTAIL = NCHUNKS - CPW * NW
RPT = 624
RTAIL = N - RPT * NS
BE = 8000


def _tile_copy(s, src_ref, dst_ref):
    row0 = s * RPT
    pltpu.sync_copy(src_ref.at[pl.ds(row0, RPT), :],
                    dst_ref.at[pl.ds(row0, RPT), :])

    @pl.when(s == NS - 1)
    def _():
        pltpu.sync_copy(src_ref.at[pl.ds(RPT * NS, RTAIL), :],
                        dst_ref.at[pl.ds(RPT * NS, RTAIL), :])


def _spmv_body(table, src, dst, zinit, out,
               src_v0, dst_v0, rows_v0, src_v1, dst_v1, rows_v1,
               src_v2, dst_v2, rows_v2, acc, sem0, sem1, sem2):
    c = lax.axis_index("c")
    s = lax.axis_index("s")
    wid = c * NS + s

    _tile_copy(s, zinit, acc)
    plsc.subcore_barrier()

    sv = (src_v0, src_v1, src_v2)
    dv = (dst_v0, dst_v1, dst_v2)
    rv = (rows_v0, rows_v1, rows_v2)
    sems = (sem0, sem1, sem2)

    def start(j, b):
        base = j * CHUNK
        pltpu.sync_copy(src.at[pl.ds(base, CHUNK)], sv[b])
        pltpu.sync_copy(dst.at[pl.ds(base, CHUNK)], dv[b])
        pltpu.async_copy(table.at[sv[b]], rv[b], sems[b])

    def finish(b):
        pltpu.make_async_copy(table.at[sv[b]], rv[b], sems[b]).wait()
        pltpu.sync_copy(rv[b], acc.at[dv[b]], add=True)

    j0 = wid * CPW
    start(j0, 0)
    start(j0 + 1, 1)
    start(j0 + 2, 2)

    def body(i, _):
        j = j0 + 3 * i
        finish(0)
        start(j + 3, 0)
        finish(1)
        start(j + 4, 1)
        finish(2)
        start(j + 5, 2)
        return 0

    lax.fori_loop(0, (CPW - 3) // 3, body, 0)
    finish(0)
    finish(1)
    finish(2)

    @pl.when(wid < NTAIL)
    def _():
        start(NW * CPW + wid, 0)
        finish(0)

    plsc.subcore_barrier()
    _tile_copy(s, acc, out.at[c])


_sc_spmv = pl.kernel(
    _spmv_body,
    out_type=jax.ShapeDtypeStruct((NC, N, D), jnp.float32),
    mesh=plsc.VectorSubcoreMesh(core_axis_name="c", subcore_axis_name="s"),
    scratch_types=[
        pltpu.VMEM((CHUNK,), jnp.int32),
        pltpu.VMEM((CHUNK,), jnp.int32),
        pltpu.VMEM((CHUNK, D), jnp.float32),
        pltpu.VMEM((CHUNK,), jnp.int32),
        pltpu.VMEM((CHUNK,), jnp.int32),
        pltpu.VMEM((CHUNK, D), jnp.float32),
        pltpu.VMEM((CHUNK,), jnp.int32),
        pltpu.VMEM((CHUNK,), jnp.int32),
        pltpu.VMEM((CHUNK, D), jnp.float32),
        pltpu.VMEM_SHARED((N, D), jnp.float32),
        pltpu.SemaphoreType.DMA,
        pltpu.SemaphoreType.DMA,
        pltpu.SemaphoreType.DMA,
    ],
)


def _edge_scatter_body(eh_full, dst, zinit, out,
                       dst_v0, rows_v0, dst_v1, rows_v1, dst_v2, rows_v2,
                       acc, sem0, sem1, sem2):
    c = lax.axis_index("c")
    s = lax.axis_index("s")
    wid = c * NS + s

    _tile_copy(s, zinit, acc)
    plsc.subcore_barrier()

    dv = (dst_v0, dst_v1, dst_v2)
    rv = (rows_v0, rows_v1, rows_v2)
    sems = (sem0, sem1, sem2)

    def start(j, b):
        base = j * CHUNK
        pltpu.sync_copy(dst.at[pl.ds(base, CHUNK)], dv[b])
        pltpu.async_copy(eh_full.at[pl.ds(base, CHUNK), :], rv[b], sems[b])

    def finish(j, b):
        base = j * CHUNK
        pltpu.make_async_copy(eh_full.at[pl.ds(base, CHUNK), :], rv[b],
                              sems[b]).wait()
        pltpu.sync_copy(rv[b], acc.at[dv[b]], add=True)

    j0 = wid * CPW
    start(j0, 0)
    start(j0 + 1, 1)
    start(j0 + 2, 2)

    def body(i, _):
        j = j0 + 3 * i
        finish(j, 0)
        start(j + 3, 0)
        finish(j + 1, 1)
        start(j + 4, 1)
        finish(j + 2, 2)
        start(j + 5, 2)
        return 0

    lax.fori_loop(0, (CPW - 3) // 3, body, 0)
    finish(j0 + CPW - 3, 0)
    finish(j0 + CPW - 2, 1)
    finish(j0 + CPW - 1, 2)

    @pl.when(wid < NTAIL)
    def _():
        start(NW * CPW + wid, 0)
        finish(NW * CPW + wid, 0)

    plsc.subcore_barrier()
    _tile_copy(s, acc, out.at[c])


_sc_edge_scatter = pl.kernel(
    _edge_scatter_body,
    out_type=jax.ShapeDtypeStruct((NC, N, D), jnp.float32),
    mesh=plsc.VectorSubcoreMesh(core_axis_name="c", subcore_axis_name="s"),
    scratch_types=[
        pltpu.VMEM((CHUNK,), jnp.int32),
        pltpu.VMEM((CHUNK, D), jnp.float32),
        pltpu.VMEM((CHUNK,), jnp.int32),
        pltpu.VMEM((CHUNK, D), jnp.float32),
        pltpu.VMEM((CHUNK,), jnp.int32),
        pltpu.VMEM((CHUNK, D), jnp.float32),
        pltpu.VMEM_SHARED((N, D), jnp.float32),
        pltpu.SemaphoreType.DMA,
        pltpu.SemaphoreType.DMA,
        pltpu.SemaphoreType.DMA,
    ],
)


_DN = (((1,), (1,)), ((), ()))


def _eh_body(e_ref, we_ref, be_ref, o_ref):
    o_ref[...] = lax.dot_general(e_ref[...], we_ref[...], _DN,
                                 preferred_element_type=jnp.float32) + be_ref[...]


def _tc_eh(e, W_edge, b_edge):
    return pl.pallas_call(
        _eh_body,
        grid=(E // BE,),
        in_specs=[pl.BlockSpec((BE, DE), lambda i: (i, 0)),
                  pl.BlockSpec((H, DE), lambda i: (0, 0)),
                  pl.BlockSpec((1, H), lambda i: (0, 0))],
        out_specs=pl.BlockSpec((BE, H), lambda i: (i, 0)),
        out_shape=jax.ShapeDtypeStruct((E, H), jnp.float32),
    )(e, W_edge, b_edge.reshape(1, H))


def _h_body(x_ref, wn_ref, bn_ref, h_ref):
    h_ref[...] = lax.dot_general(x_ref[...], wn_ref[...], _DN,
                                 preferred_element_type=jnp.float32) + bn_ref[...]


def _tc_h(x, W_node, b_node):
    return pl.pallas_call(
        _h_body,
        out_shape=jax.ShapeDtypeStruct((N, H), jnp.float32),
    )(x, W_node, b_node.reshape(1, H))


def _layer0_body(p_ref, pe_ref, tin_ref, w1_ref, b1_ref, g1_ref, be1_ref,
                 w2_ref, b2_ref, hs_ref, tout_ref, eh_ref):
    eh = pe_ref[0] + pe_ref[1]
    eh_ref[...] = eh
    aggr = p_ref[0] + p_ref[1] + eh
    z = lax.dot_general(aggr, w1_ref[...], _DN,
                        preferred_element_type=jnp.float32) + b1_ref[...]
    m = jnp.mean(z, axis=0, keepdims=True)
    v = jnp.mean((z - m) ** 2, axis=0, keepdims=True)
    z = g1_ref[...] * (z - m) / jnp.sqrt(v + 1e-5) + be1_ref[...]
    z = jnp.maximum(z, 0.0)
    z = lax.dot_general(z, w2_ref[...], _DN,
                        preferred_element_type=jnp.float32) + b2_ref[...]
    hs = jnp.where(z > 0, z, 0.2 * z)
    hs_ref[...] = hs
    tout_ref[...] = tin_ref[...] + hs


def _tc_layer0(p, pe, total, W1l, b1l, g1l, be1l, W2l, b2l):
    return pl.pallas_call(
        _layer0_body,
        out_shape=[jax.ShapeDtypeStruct((N, H), jnp.float32),
                   jax.ShapeDtypeStruct((N, H), jnp.float32),
                   jax.ShapeDtypeStruct((N, H), jnp.float32)],
    )(p, pe, total, W1l, b1l.reshape(1, 2 * H), g1l.reshape(1, 2 * H),
      be1l.reshape(1, 2 * H), W2l, b2l.reshape(1, H))


def _layer_body(p_ref, eh_ref, tin_ref, w1_ref, b1_ref, g1_ref, be1_ref,
                w2_ref, b2_ref, hs_ref, tout_ref):
    aggr = p_ref[0] + p_ref[1] + eh_ref[...]
    z = lax.dot_general(aggr, w1_ref[...], _DN,
                        preferred_element_type=jnp.float32) + b1_ref[...]
    m = jnp.mean(z, axis=0, keepdims=True)
    v = jnp.mean((z - m) ** 2, axis=0, keepdims=True)
    z = g1_ref[...] * (z - m) / jnp.sqrt(v + 1e-5) + be1_ref[...]
    z = jnp.maximum(z, 0.0)
    z = lax.dot_general(z, w2_ref[...], _DN,
                        preferred_element_type=jnp.float32) + b2_ref[...]
    hs = jnp.where(z > 0, z, 0.2 * z)
    hs_ref[...] = hs
    tout_ref[...] = tin_ref[...] + hs


def _tc_layer(p, eh, total, W1l, b1l, g1l, be1l, W2l, b2l):
    return pl.pallas_call(
        _layer_body,
        out_shape=[jax.ShapeDtypeStruct((N, H), jnp.float32),
                   jax.ShapeDtypeStruct((N, H), jnp.float32)],
    )(p, eh, total, W1l, b1l.reshape(1, 2 * H), g1l.reshape(1, 2 * H),
      be1l.reshape(1, 2 * H), W2l, b2l.reshape(1, H))


def _head_body(t_ref, wo1_ref, bo1_ref, go_ref, beo_ref, ap_ref, wo2_ref,
               bo2_ref, o_ref):
    z = lax.dot_general(t_ref[...], wo1_ref[...], _DN,
                        preferred_element_type=jnp.float32) + bo1_ref[...]
    m = jnp.mean(z, axis=0, keepdims=True)
    v = jnp.mean((z - m) ** 2, axis=0, keepdims=True)
    z = go_ref[...] * (z - m) / jnp.sqrt(v + 1e-5) + beo_ref[...]
    z = jnp.where(z > 0, z, ap_ref[...] * z)
    o_ref[...] = lax.dot_general(z, wo2_ref[...], _DN,
                                 preferred_element_type=jnp.float32) + bo2_ref[...]


def _tc_head(total, Wo1, bo1, go, beo, a_prelu, Wo2, bo2):
    return pl.pallas_call(
        _head_body,
        out_shape=jax.ShapeDtypeStruct((N, C), jnp.float32),
    )(total, Wo1, bo1.reshape(1, 2 * H), go.reshape(1, 2 * H),
      beo.reshape(1, 2 * H), a_prelu.reshape(1, 1), Wo2, bo2.reshape(1, C))


def kernel(x, edge_index, e, W_node, b_node, W_edge, b_edge, W1, b1, g1, be1,
           W2, b2, Wo1, bo1, go, beo, a_prelu, Wo2, bo2):
    src = edge_index[0]
    dst = edge_index[1]

    zin_d = jnp.zeros((N, D), jnp.float32)

    h = _tc_h(x, W_node, b_node)
    p = _sc_spmv(h, src, dst, zin_d)
    eh_full = _tc_eh(e, W_edge, b_edge)
    pe = _sc_edge_scatter(eh_full, dst, zin_d)

    hs, total, eh = _tc_layer0(p, pe, h, W1[0], b1[0], g1[0], be1[0],
                               W2[0], b2[0])
    for l in range(1, L):
        p = _sc_spmv(hs, src, dst, zin_d)
        hs, total = _tc_layer(p, eh, total, W1[l], b1[l], g1[l], be1[l],
                              W2[l], b2[l])

    return _tc_head(total, Wo1, bo1, go, beo, a_prelu, Wo2, bo2)
